# baseline (speedup 1.0000x reference)
import functools

import jax
import jax.numpy as jnp
from jax import lax
from jax.experimental import pallas as pl
from jax.experimental.pallas import tpu as pltpu
from jax.experimental.pallas import tpu_sc as plsc

N_NODES = 10000
N_PAD = 10240
IN_DIM = 128
HID_DIM = 64
OUT_DIM = 10
NC = 2
NS = 16
NW = NC * NS
CHUNK = 125
NBUF = 4
ROWS_PER_TILE = N_PAD // NS


@functools.lru_cache(maxsize=None)
def _mesh():
    return plsc.VectorSubcoreMesh(core_axis_name="c", subcore_axis_name="s",
                                  num_cores=NC, num_subcores=NS)


def _deg_body(e2_hbm, out_hbm, dstv, ones_v, zv, deg_sh):
    cid = lax.axis_index("c")
    sid = lax.axis_index("s")
    wid = sid * NC + cid
    n_chunks = e2_hbm.shape[1] // NW

    def zstep(i, carry):
        zv[pl.ds(i * 16, 16)] = jnp.zeros((16,), jnp.float32)
        return carry
    lax.fori_loop(0, ROWS_PER_TILE // 16, zstep, 0)
    pltpu.sync_copy(zv, deg_sh.at[pl.ds(sid * ROWS_PER_TILE, ROWS_PER_TILE)])

    def ostep(i, carry):
        ones_v[pl.ds(i * 16, 16)] = jnp.ones((16,), jnp.float32)
        return carry
    lax.fori_loop(0, 8, ostep, 0)

    plsc.subcore_barrier()

    pltpu.sync_copy(e2_hbm.at[2, pl.ds(wid * n_chunks, n_chunks)], dstv)

    def step(j, carry):
        pltpu.sync_copy(ones_v.at[pl.ds(0, CHUNK)], deg_sh.at[dstv.at[j]],
                        add=True)
        return carry
    lax.fori_loop(0, n_chunks, step, 0)

    plsc.subcore_barrier()
    pltpu.sync_copy(deg_sh.at[pl.ds(sid * ROWS_PER_TILE, ROWS_PER_TILE)],
                    out_hbm.at[cid, pl.ds(sid * ROWS_PER_TILE, ROWS_PER_TILE)])


def _make_deg(n_chunks_total):
    return pl.kernel(
        _deg_body,
        out_type=jax.ShapeDtypeStruct((NC, N_PAD), jnp.float32),
        mesh=_mesh(),
        scratch_types=[
            pltpu.VMEM((n_chunks_total // NW, CHUNK), jnp.int32),
            pltpu.VMEM((128,), jnp.float32),
            pltpu.VMEM((ROWS_PER_TILE,), jnp.float32),
            pltpu.VMEM_SHARED((N_PAD,), jnp.float32),
        ],
        compiler_params=pltpu.CompilerParams(use_tc_tiling_on_sc=False),
    )


HHID = HID_DIM // 2


def _agg_body(hp_hbm, e3_hbm, out_hbm, srcv, dstv,
              rows0, rows1, rows2, rows3, zv, acc_sh,
              sg0, sg1, sg2, sg3, ss0, ss1, ss2, ss3):
    rows = [rows0, rows1, rows2, rows3]
    sg = [sg0, sg1, sg2, sg3]
    ss = [ss0, ss1, ss2, ss3]
    cid = lax.axis_index("c")
    sid = lax.axis_index("s")
    n_chunks = e3_hbm.shape[1] // NS
    n_groups = n_chunks // NBUF

    def zstep(i, carry):
        r = i // 2
        c = i % 2
        zv[r, pl.ds(c * 16, 16)] = jnp.zeros((16,), jnp.float32)
        return carry
    lax.fori_loop(0, 128 * 2, zstep, 0)

    def zcopy(k, carry):
        pltpu.sync_copy(
            zv, acc_sh.at[pl.ds(sid * ROWS_PER_TILE + k * 128, 128)])
        return carry
    lax.fori_loop(0, ROWS_PER_TILE // 128, zcopy, 0)

    plsc.subcore_barrier()

    pltpu.sync_copy(e3_hbm.at[cid, pl.ds(sid * n_chunks, n_chunks)], srcv)
    pltpu.sync_copy(e3_hbm.at[2, pl.ds(sid * n_chunks, n_chunks)], dstv)

    for b in range(NBUF):
        pltpu.async_copy(hp_hbm.at[srcv.at[b]], rows[b], sg[b])

    def group(g, carry):
        for b in range(NBUF):
            c = NBUF * g + b
            pltpu.make_async_copy(hp_hbm.at[srcv.at[c]], rows[b],
                                  sg[b]).wait()
            pltpu.async_copy(rows[b], acc_sh.at[dstv.at[c]], ss[b], add=True)
        for b in range(NBUF):
            c = NBUF * g + b

            @pl.when(g < n_groups - 1)
            def _next():
                pltpu.make_async_copy(rows[b], acc_sh.at[dstv.at[c]],
                                      ss[b]).wait()
                pltpu.async_copy(hp_hbm.at[srcv.at[c + NBUF]], rows[b], sg[b])
        return carry
    lax.fori_loop(0, n_groups, group, 0)

    for b in range(NBUF):
        pltpu.make_async_copy(rows[b],
                              acc_sh.at[dstv.at[n_chunks - NBUF + b]],
                              ss[b]).wait()

    plsc.subcore_barrier()
    pltpu.sync_copy(
        acc_sh.at[pl.ds(sid * ROWS_PER_TILE, ROWS_PER_TILE)],
        out_hbm.at[pl.ds(sid * ROWS_PER_TILE, ROWS_PER_TILE),
                   pl.ds(cid * HHID, HHID)])


def _make_agg(n_chunks_total):
    return pl.kernel(
        _agg_body,
        out_type=jax.ShapeDtypeStruct((N_PAD, HID_DIM), jnp.float32),
        mesh=_mesh(),
        scratch_types=[
            pltpu.VMEM((n_chunks_total // NS, CHUNK), jnp.int32),
            pltpu.VMEM((n_chunks_total // NS, CHUNK), jnp.int32),
            pltpu.VMEM((CHUNK, HHID), jnp.float32),
            pltpu.VMEM((CHUNK, HHID), jnp.float32),
            pltpu.VMEM((CHUNK, HHID), jnp.float32),
            pltpu.VMEM((CHUNK, HHID), jnp.float32),
            pltpu.VMEM((128, HHID), jnp.float32),
            pltpu.VMEM_SHARED((N_PAD, HHID), jnp.float32),
            pltpu.SemaphoreType.DMA,
            pltpu.SemaphoreType.DMA,
            pltpu.SemaphoreType.DMA,
            pltpu.SemaphoreType.DMA,
            pltpu.SemaphoreType.DMA,
            pltpu.SemaphoreType.DMA,
            pltpu.SemaphoreType.DMA,
            pltpu.SemaphoreType.DMA,
        ],
        compiler_params=pltpu.CompilerParams(use_tc_tiling_on_sc=False),
    )


_BR = 2048


def _dinv(deg_ref):
    deg = deg_ref[0] + deg_ref[1] + 1.0
    return lax.rsqrt(deg)[:, None]


def _tc1_body(deg_ref, x_ref, w1_ref, hp1_ref):
    h = jnp.dot(x_ref[...], w1_ref[...], preferred_element_type=jnp.float32)
    hp1_ref[...] = h * _dinv(deg_ref)


def _tc2_body(s_ref, hp1_ref, deg_ref, b1_ref, w2_ref, hp2_ref):
    dinv = _dinv(deg_ref)
    s = s_ref[...] + hp1_ref[...]
    out1 = jnp.maximum(dinv * s + b1_ref[...], 0.0)
    h2 = jnp.dot(out1, w2_ref[...], preferred_element_type=jnp.float32)
    hp2_ref[...] = h2 * dinv


def _tc3_body(s_ref, hp2_ref, deg_ref, b2_ref, wf1_ref, bf1_ref,
              wf2_ref, bf2_ref, out_ref):
    dinv = _dinv(deg_ref)
    s = s_ref[...] + hp2_ref[...]
    out2 = jnp.maximum(dinv * s + b2_ref[...], 0.0)
    h3 = jnp.maximum(
        jnp.dot(out2, wf1_ref[...], preferred_element_type=jnp.float32)
        + bf1_ref[...], 0.0)
    logits = (jnp.dot(h3, wf2_ref[...], preferred_element_type=jnp.float32)
              + bf2_ref[...])
    m = jnp.max(logits, axis=1, keepdims=True)
    e = jnp.exp(logits - m)
    out_ref[...] = e / jnp.sum(e, axis=1, keepdims=True)


def _row_spec(cols):
    return pl.BlockSpec((_BR, cols), lambda i: (i, 0))


def _pad_spec(cols):
    return pl.BlockSpec((NC, _BR, cols), lambda i: (0, i, 0))


_deg_spec = pl.BlockSpec((NC, _BR), lambda i: (0, i))


def _full_spec(shape):
    nd = len(shape)
    return pl.BlockSpec(shape, lambda i: (0,) * nd)


_GRID = (N_PAD // _BR,)

_tc1 = pl.pallas_call(
    _tc1_body,
    grid=_GRID,
    in_specs=[
        _deg_spec,
        _row_spec(IN_DIM),
        _full_spec((IN_DIM, HID_DIM)),
    ],
    out_specs=_row_spec(HID_DIM),
    out_shape=jax.ShapeDtypeStruct((N_PAD, HID_DIM), jnp.float32),
)

_tc2 = pl.pallas_call(
    _tc2_body,
    grid=_GRID,
    in_specs=[
        _row_spec(HID_DIM),
        _row_spec(HID_DIM),
        _deg_spec,
        _full_spec((1, HID_DIM)),
        _full_spec((HID_DIM, HID_DIM)),
    ],
    out_specs=_row_spec(HID_DIM),
    out_shape=jax.ShapeDtypeStruct((N_PAD, HID_DIM), jnp.float32),
)

_tc3 = pl.pallas_call(
    _tc3_body,
    grid=_GRID,
    in_specs=[
        _row_spec(HID_DIM),
        _row_spec(HID_DIM),
        _deg_spec,
        _full_spec((1, HID_DIM)),
        _full_spec((HID_DIM, HID_DIM)),
        _full_spec((1, HID_DIM)),
        _full_spec((HID_DIM, OUT_DIM)),
        _full_spec((1, OUT_DIM)),
    ],
    out_specs=_row_spec(OUT_DIM),
    out_shape=jax.ShapeDtypeStruct((N_PAD, OUT_DIM), jnp.float32),
)


@jax.jit
def kernel(x, edge_index, W1, b1, W2, b2, Wf1, bf1, Wf2, bf2):
    n_edges = edge_index.shape[1]
    n_chunks_total = n_edges // CHUNK
    ei = edge_index.astype(jnp.int32)
    e3 = jnp.stack([2 * ei[0], 2 * ei[0] + 1, ei[1]]
                   ).reshape(3, n_chunks_total, CHUNK)

    degp = _make_deg(n_chunks_total)(e3)

    xp = jnp.pad(x, ((0, N_PAD - N_NODES), (0, 0)))
    hp1 = _tc1(degp, xp, W1)

    agg = _make_agg(n_chunks_total)
    s1 = agg(hp1.reshape(2 * N_PAD, HHID), e3)
    hp2 = _tc2(s1, hp1, degp, b1.reshape(1, HID_DIM), W2)

    s2 = agg(hp2.reshape(2 * N_PAD, HHID), e3)
    out = _tc3(s2, hp2, degp, b2.reshape(1, HID_DIM), Wf1,
               bf1.reshape(1, HID_DIM), Wf2, bf2.reshape(1, OUT_DIM))
    return out[:N_NODES]


# speedup vs baseline: 1.0735x; 1.0735x over previous
import functools

import jax
import jax.numpy as jnp
from jax import lax
from jax.experimental import pallas as pl
from jax.experimental.pallas import tpu as pltpu
from jax.experimental.pallas import tpu_sc as plsc

N_NODES = 10000
N_PAD = 10240
IN_DIM = 128
HID_DIM = 64
OUT_DIM = 10
NC = 2
NS = 16
NW = NC * NS
CHUNK = 125
NBUF = 4
ROWS_PER_TILE = N_PAD // NS


@functools.lru_cache(maxsize=None)
def _mesh():
    return plsc.VectorSubcoreMesh(core_axis_name="c", subcore_axis_name="s",
                                  num_cores=NC, num_subcores=NS)


def _deg_body(e2_hbm, out_hbm, dstv, ones_v, zv, deg_sh):
    cid = lax.axis_index("c")
    sid = lax.axis_index("s")
    wid = sid * NC + cid
    n_chunks = e2_hbm.shape[1] // NW

    def zstep(i, carry):
        zv[pl.ds(i * 16, 16)] = jnp.zeros((16,), jnp.float32)
        return carry
    lax.fori_loop(0, ROWS_PER_TILE // 16, zstep, 0)
    pltpu.sync_copy(zv, deg_sh.at[pl.ds(sid * ROWS_PER_TILE, ROWS_PER_TILE)])

    def ostep(i, carry):
        ones_v[pl.ds(i * 16, 16)] = jnp.ones((16,), jnp.float32)
        return carry
    lax.fori_loop(0, 8, ostep, 0)

    plsc.subcore_barrier()

    pltpu.sync_copy(e2_hbm.at[1, pl.ds(wid * n_chunks, n_chunks)], dstv)

    def step(j, carry):
        pltpu.sync_copy(ones_v.at[pl.ds(0, CHUNK)], deg_sh.at[dstv.at[j]],
                        add=True)
        return carry
    lax.fori_loop(0, n_chunks, step, 0)

    plsc.subcore_barrier()
    pltpu.sync_copy(deg_sh.at[pl.ds(sid * ROWS_PER_TILE, ROWS_PER_TILE)],
                    out_hbm.at[cid, pl.ds(sid * ROWS_PER_TILE, ROWS_PER_TILE)])


def _make_deg(n_chunks_total):
    return pl.kernel(
        _deg_body,
        out_type=jax.ShapeDtypeStruct((NC, N_PAD), jnp.float32),
        mesh=_mesh(),
        scratch_types=[
            pltpu.VMEM((n_chunks_total // NW, CHUNK), jnp.int32),
            pltpu.VMEM((128,), jnp.float32),
            pltpu.VMEM((ROWS_PER_TILE,), jnp.float32),
            pltpu.VMEM_SHARED((N_PAD,), jnp.float32),
        ],
        compiler_params=pltpu.CompilerParams(use_tc_tiling_on_sc=False),
    )


def _agg_body(hp_hbm, e2_hbm, out_hbm, srcv, dstv,
              rows0, rows1, rows2, rows3, zv, acc_sh,
              sg0, sg1, sg2, sg3, ss0, ss1, ss2, ss3):
    rows = [rows0, rows1, rows2, rows3]
    sg = [sg0, sg1, sg2, sg3]
    ss = [ss0, ss1, ss2, ss3]
    cid = lax.axis_index("c")
    sid = lax.axis_index("s")
    wid = sid * NC + cid
    n_chunks = e2_hbm.shape[1] // NW
    n_groups = n_chunks // NBUF

    def zstep(i, carry):
        r = i // 4
        c = i % 4
        zv[r, pl.ds(c * 16, 16)] = jnp.zeros((16,), jnp.float32)
        return carry
    lax.fori_loop(0, 128 * 4, zstep, 0)

    def zcopy(k, carry):
        pltpu.sync_copy(
            zv, acc_sh.at[pl.ds(sid * ROWS_PER_TILE + k * 128, 128)])
        return carry
    lax.fori_loop(0, ROWS_PER_TILE // 128, zcopy, 0)

    plsc.subcore_barrier()

    pltpu.sync_copy(e2_hbm.at[0, pl.ds(wid * n_chunks, n_chunks)], srcv)
    pltpu.sync_copy(e2_hbm.at[1, pl.ds(wid * n_chunks, n_chunks)], dstv)

    for b in range(NBUF):
        pltpu.async_copy(hp_hbm.at[srcv.at[b]], rows[b], sg[b])

    def group(g, carry):
        for b in range(NBUF):
            c = NBUF * g + b
            pltpu.make_async_copy(hp_hbm.at[srcv.at[c]], rows[b],
                                  sg[b]).wait()
            pltpu.async_copy(rows[b], acc_sh.at[dstv.at[c]], ss[b], add=True)
        for b in range(NBUF):
            c = NBUF * g + b

            @pl.when(g < n_groups - 1)
            def _next():
                pltpu.make_async_copy(rows[b], acc_sh.at[dstv.at[c]],
                                      ss[b]).wait()
                pltpu.async_copy(hp_hbm.at[srcv.at[c + NBUF]], rows[b], sg[b])
        return carry
    lax.fori_loop(0, n_groups, group, 0)

    for b in range(NBUF):
        pltpu.make_async_copy(rows[b],
                              acc_sh.at[dstv.at[n_chunks - NBUF + b]],
                              ss[b]).wait()

    plsc.subcore_barrier()
    pltpu.sync_copy(
        acc_sh.at[pl.ds(sid * ROWS_PER_TILE, ROWS_PER_TILE)],
        out_hbm.at[cid, pl.ds(sid * ROWS_PER_TILE, ROWS_PER_TILE)])


def _make_agg(n_chunks_total):
    return pl.kernel(
        _agg_body,
        out_type=jax.ShapeDtypeStruct((NC, N_PAD, HID_DIM), jnp.float32),
        mesh=_mesh(),
        scratch_types=[
            pltpu.VMEM((n_chunks_total // NW, CHUNK), jnp.int32),
            pltpu.VMEM((n_chunks_total // NW, CHUNK), jnp.int32),
            pltpu.VMEM((CHUNK, HID_DIM), jnp.float32),
            pltpu.VMEM((CHUNK, HID_DIM), jnp.float32),
            pltpu.VMEM((CHUNK, HID_DIM), jnp.float32),
            pltpu.VMEM((CHUNK, HID_DIM), jnp.float32),
            pltpu.VMEM((128, HID_DIM), jnp.float32),
            pltpu.VMEM_SHARED((N_PAD, HID_DIM), jnp.float32),
            pltpu.SemaphoreType.DMA,
            pltpu.SemaphoreType.DMA,
            pltpu.SemaphoreType.DMA,
            pltpu.SemaphoreType.DMA,
            pltpu.SemaphoreType.DMA,
            pltpu.SemaphoreType.DMA,
            pltpu.SemaphoreType.DMA,
            pltpu.SemaphoreType.DMA,
        ],
        compiler_params=pltpu.CompilerParams(use_tc_tiling_on_sc=False),
    )


_BR = 2048


def _dinv(deg_ref):
    deg = deg_ref[0] + deg_ref[1] + 1.0
    return lax.rsqrt(deg)[:, None]


def _tc1_body(deg_ref, x_ref, w1_ref, hp1_ref):
    h = jnp.dot(x_ref[...], w1_ref[...], preferred_element_type=jnp.float32)
    hp1_ref[...] = h * _dinv(deg_ref)


def _tc2_body(s_ref, hp1_ref, deg_ref, b1_ref, w2_ref, hp2_ref):
    dinv = _dinv(deg_ref)
    s = s_ref[0] + s_ref[1] + hp1_ref[...]
    out1 = jnp.maximum(dinv * s + b1_ref[...], 0.0)
    h2 = jnp.dot(out1, w2_ref[...], preferred_element_type=jnp.float32)
    hp2_ref[...] = h2 * dinv


def _tc3_body(s_ref, hp2_ref, deg_ref, b2_ref, wf1_ref, bf1_ref,
              wf2_ref, bf2_ref, out_ref):
    dinv = _dinv(deg_ref)
    s = s_ref[0] + s_ref[1] + hp2_ref[...]
    out2 = jnp.maximum(dinv * s + b2_ref[...], 0.0)
    h3 = jnp.maximum(
        jnp.dot(out2, wf1_ref[...], preferred_element_type=jnp.float32)
        + bf1_ref[...], 0.0)
    logits = (jnp.dot(h3, wf2_ref[...], preferred_element_type=jnp.float32)
              + bf2_ref[...])
    m = jnp.max(logits, axis=1, keepdims=True)
    e = jnp.exp(logits - m)
    out_ref[...] = e / jnp.sum(e, axis=1, keepdims=True)


def _row_spec(cols):
    return pl.BlockSpec((_BR, cols), lambda i: (i, 0))


def _pad_spec(cols):
    return pl.BlockSpec((NC, _BR, cols), lambda i: (0, i, 0))


_deg_spec = pl.BlockSpec((NC, _BR), lambda i: (0, i))


def _full_spec(shape):
    nd = len(shape)
    return pl.BlockSpec(shape, lambda i: (0,) * nd)


_GRID = (N_PAD // _BR,)

_tc1 = pl.pallas_call(
    _tc1_body,
    grid=_GRID,
    in_specs=[
        _deg_spec,
        _row_spec(IN_DIM),
        _full_spec((IN_DIM, HID_DIM)),
    ],
    out_specs=_row_spec(HID_DIM),
    out_shape=jax.ShapeDtypeStruct((N_PAD, HID_DIM), jnp.float32),
)

_tc2 = pl.pallas_call(
    _tc2_body,
    grid=_GRID,
    in_specs=[
        _pad_spec(HID_DIM),
        _row_spec(HID_DIM),
        _deg_spec,
        _full_spec((1, HID_DIM)),
        _full_spec((HID_DIM, HID_DIM)),
    ],
    out_specs=_row_spec(HID_DIM),
    out_shape=jax.ShapeDtypeStruct((N_PAD, HID_DIM), jnp.float32),
)

_tc3 = pl.pallas_call(
    _tc3_body,
    grid=_GRID,
    in_specs=[
        _pad_spec(HID_DIM),
        _row_spec(HID_DIM),
        _deg_spec,
        _full_spec((1, HID_DIM)),
        _full_spec((HID_DIM, HID_DIM)),
        _full_spec((1, HID_DIM)),
        _full_spec((HID_DIM, OUT_DIM)),
        _full_spec((1, OUT_DIM)),
    ],
    out_specs=_row_spec(OUT_DIM),
    out_shape=jax.ShapeDtypeStruct((N_PAD, OUT_DIM), jnp.float32),
)


@jax.jit
def kernel(x, edge_index, W1, b1, W2, b2, Wf1, bf1, Wf2, bf2):
    n_edges = edge_index.shape[1]
    n_chunks_total = n_edges // CHUNK
    e2 = edge_index.astype(jnp.int32).reshape(2, n_chunks_total, CHUNK)

    degp = _make_deg(n_chunks_total)(e2)

    xp = jnp.pad(x, ((0, N_PAD - N_NODES), (0, 0)))
    hp1 = _tc1(degp, xp, W1)

    agg = _make_agg(n_chunks_total)
    s1 = agg(hp1, e2)
    hp2 = _tc2(s1, hp1, degp, b1.reshape(1, HID_DIM), W2)

    s2 = agg(hp2, e2)
    out = _tc3(s2, hp2, degp, b2.reshape(1, HID_DIM), Wf1,
               bf1.reshape(1, HID_DIM), Wf2, bf2.reshape(1, OUT_DIM))
    return out[:N_NODES]


# speedup vs baseline: 1.0818x; 1.0078x over previous
import functools

import jax
import jax.numpy as jnp
from jax import lax
from jax.experimental import pallas as pl
from jax.experimental.pallas import tpu as pltpu
from jax.experimental.pallas import tpu_sc as plsc

N_NODES = 10000
N_PAD = 10240
IN_DIM = 128
HID_DIM = 64
OUT_DIM = 10
NC = 2
NS = 16
NW = NC * NS
CHUNK = 125
NBUF = 5
ROWS_PER_TILE = N_PAD // NS


@functools.lru_cache(maxsize=None)
def _mesh():
    return plsc.VectorSubcoreMesh(core_axis_name="c", subcore_axis_name="s",
                                  num_cores=NC, num_subcores=NS)


def _deg_body(e2_hbm, out_hbm, dstv, ones_v, zv, deg_sh):
    cid = lax.axis_index("c")
    sid = lax.axis_index("s")
    wid = sid * NC + cid
    n_chunks = e2_hbm.shape[1] // NW

    def zstep(i, carry):
        zv[pl.ds(i * 16, 16)] = jnp.zeros((16,), jnp.float32)
        return carry
    lax.fori_loop(0, ROWS_PER_TILE // 16, zstep, 0)
    pltpu.sync_copy(zv, deg_sh.at[pl.ds(sid * ROWS_PER_TILE, ROWS_PER_TILE)])

    def ostep(i, carry):
        ones_v[pl.ds(i * 16, 16)] = jnp.ones((16,), jnp.float32)
        return carry
    lax.fori_loop(0, 8, ostep, 0)

    plsc.subcore_barrier()

    pltpu.sync_copy(e2_hbm.at[1, pl.ds(wid * n_chunks, n_chunks)], dstv)

    def step(j, carry):
        pltpu.sync_copy(ones_v.at[pl.ds(0, CHUNK)], deg_sh.at[dstv.at[j]],
                        add=True)
        return carry
    lax.fori_loop(0, n_chunks, step, 0)

    plsc.subcore_barrier()
    pltpu.sync_copy(deg_sh.at[pl.ds(sid * ROWS_PER_TILE, ROWS_PER_TILE)],
                    out_hbm.at[cid, pl.ds(sid * ROWS_PER_TILE, ROWS_PER_TILE)])


def _make_deg(n_chunks_total):
    return pl.kernel(
        _deg_body,
        out_type=jax.ShapeDtypeStruct((NC, N_PAD), jnp.float32),
        mesh=_mesh(),
        scratch_types=[
            pltpu.VMEM((n_chunks_total // NW, CHUNK), jnp.int32),
            pltpu.VMEM((128,), jnp.float32),
            pltpu.VMEM((ROWS_PER_TILE,), jnp.float32),
            pltpu.VMEM_SHARED((N_PAD,), jnp.float32),
        ],
        compiler_params=pltpu.CompilerParams(use_tc_tiling_on_sc=False),
    )


def _agg_body(hp_hbm, e2_hbm, out_hbm, srcv, dstv, *rest):
    rows = list(rest[:NBUF])
    zv = rest[NBUF]
    acc_sh = rest[NBUF + 1]
    sg = list(rest[NBUF + 2:NBUF + 2 + NBUF])
    ss = list(rest[NBUF + 2 + NBUF:])
    cid = lax.axis_index("c")
    sid = lax.axis_index("s")
    wid = sid * NC + cid
    n_chunks = e2_hbm.shape[1] // NW
    n_groups = n_chunks // NBUF

    def zstep(i, carry):
        r = i // 4
        c = i % 4
        zv[r, pl.ds(c * 16, 16)] = jnp.zeros((16,), jnp.float32)
        return carry
    lax.fori_loop(0, 128 * 4, zstep, 0)

    def zcopy(k, carry):
        pltpu.sync_copy(
            zv, acc_sh.at[pl.ds(sid * ROWS_PER_TILE + k * 128, 128)])
        return carry
    lax.fori_loop(0, ROWS_PER_TILE // 128, zcopy, 0)

    plsc.subcore_barrier()

    pltpu.sync_copy(e2_hbm.at[0, pl.ds(wid * n_chunks, n_chunks)], srcv)
    pltpu.sync_copy(e2_hbm.at[1, pl.ds(wid * n_chunks, n_chunks)], dstv)

    for b in range(NBUF):
        pltpu.async_copy(hp_hbm.at[srcv.at[b]], rows[b], sg[b])

    def group(g, carry):
        for b in range(NBUF):
            c = NBUF * g + b
            pltpu.make_async_copy(hp_hbm.at[srcv.at[c]], rows[b],
                                  sg[b]).wait()
            pltpu.async_copy(rows[b], acc_sh.at[dstv.at[c]], ss[b], add=True)
        for b in range(NBUF):
            c = NBUF * g + b

            @pl.when(g < n_groups - 1)
            def _next():
                pltpu.make_async_copy(rows[b], acc_sh.at[dstv.at[c]],
                                      ss[b]).wait()
                pltpu.async_copy(hp_hbm.at[srcv.at[c + NBUF]], rows[b], sg[b])
        return carry
    lax.fori_loop(0, n_groups, group, 0)

    for b in range(NBUF):
        pltpu.make_async_copy(rows[b],
                              acc_sh.at[dstv.at[n_chunks - NBUF + b]],
                              ss[b]).wait()

    plsc.subcore_barrier()
    pltpu.sync_copy(
        acc_sh.at[pl.ds(sid * ROWS_PER_TILE, ROWS_PER_TILE)],
        out_hbm.at[cid, pl.ds(sid * ROWS_PER_TILE, ROWS_PER_TILE)])


def _make_agg(n_chunks_total):
    return pl.kernel(
        _agg_body,
        out_type=jax.ShapeDtypeStruct((NC, N_PAD, HID_DIM), jnp.float32),
        mesh=_mesh(),
        scratch_types=(
            [pltpu.VMEM((n_chunks_total // NW, CHUNK), jnp.int32)] * 2
            + [pltpu.VMEM((CHUNK, HID_DIM), jnp.float32)] * NBUF
            + [pltpu.VMEM((128, HID_DIM), jnp.float32),
               pltpu.VMEM_SHARED((N_PAD, HID_DIM), jnp.float32)]
            + [pltpu.SemaphoreType.DMA] * (2 * NBUF)
        ),
        compiler_params=pltpu.CompilerParams(use_tc_tiling_on_sc=False),
    )


_BR = 2048


def _dinv(deg_ref):
    deg = deg_ref[0] + deg_ref[1] + 1.0
    return lax.rsqrt(deg)[:, None]


def _tc1_body(deg_ref, x_ref, w1_ref, hp1_ref):
    h = jnp.dot(x_ref[...], w1_ref[...], preferred_element_type=jnp.float32)
    hp1_ref[...] = h * _dinv(deg_ref)


def _tc2_body(s_ref, hp1_ref, deg_ref, b1_ref, w2_ref, hp2_ref):
    dinv = _dinv(deg_ref)
    s = s_ref[0] + s_ref[1] + hp1_ref[...]
    out1 = jnp.maximum(dinv * s + b1_ref[...], 0.0)
    h2 = jnp.dot(out1, w2_ref[...], preferred_element_type=jnp.float32)
    hp2_ref[...] = h2 * dinv


def _tc3_body(s_ref, hp2_ref, deg_ref, b2_ref, wf1_ref, bf1_ref,
              wf2_ref, bf2_ref, out_ref):
    dinv = _dinv(deg_ref)
    s = s_ref[0] + s_ref[1] + hp2_ref[...]
    out2 = jnp.maximum(dinv * s + b2_ref[...], 0.0)
    h3 = jnp.maximum(
        jnp.dot(out2, wf1_ref[...], preferred_element_type=jnp.float32)
        + bf1_ref[...], 0.0)
    logits = (jnp.dot(h3, wf2_ref[...], preferred_element_type=jnp.float32)
              + bf2_ref[...])
    m = jnp.max(logits, axis=1, keepdims=True)
    e = jnp.exp(logits - m)
    out_ref[...] = e / jnp.sum(e, axis=1, keepdims=True)


def _row_spec(cols):
    return pl.BlockSpec((_BR, cols), lambda i: (i, 0))


def _pad_spec(cols):
    return pl.BlockSpec((NC, _BR, cols), lambda i: (0, i, 0))


_deg_spec = pl.BlockSpec((NC, _BR), lambda i: (0, i))


def _full_spec(shape):
    nd = len(shape)
    return pl.BlockSpec(shape, lambda i: (0,) * nd)


_GRID = (N_PAD // _BR,)

_tc1 = pl.pallas_call(
    _tc1_body,
    grid=_GRID,
    in_specs=[
        _deg_spec,
        _row_spec(IN_DIM),
        _full_spec((IN_DIM, HID_DIM)),
    ],
    out_specs=_row_spec(HID_DIM),
    out_shape=jax.ShapeDtypeStruct((N_PAD, HID_DIM), jnp.float32),
)

_tc2 = pl.pallas_call(
    _tc2_body,
    grid=_GRID,
    in_specs=[
        _pad_spec(HID_DIM),
        _row_spec(HID_DIM),
        _deg_spec,
        _full_spec((1, HID_DIM)),
        _full_spec((HID_DIM, HID_DIM)),
    ],
    out_specs=_row_spec(HID_DIM),
    out_shape=jax.ShapeDtypeStruct((N_PAD, HID_DIM), jnp.float32),
)

_tc3 = pl.pallas_call(
    _tc3_body,
    grid=_GRID,
    in_specs=[
        _pad_spec(HID_DIM),
        _row_spec(HID_DIM),
        _deg_spec,
        _full_spec((1, HID_DIM)),
        _full_spec((HID_DIM, HID_DIM)),
        _full_spec((1, HID_DIM)),
        _full_spec((HID_DIM, OUT_DIM)),
        _full_spec((1, OUT_DIM)),
    ],
    out_specs=_row_spec(OUT_DIM),
    out_shape=jax.ShapeDtypeStruct((N_PAD, OUT_DIM), jnp.float32),
)


@jax.jit
def kernel(x, edge_index, W1, b1, W2, b2, Wf1, bf1, Wf2, bf2):
    n_edges = edge_index.shape[1]
    n_chunks_total = n_edges // CHUNK
    e2 = edge_index.astype(jnp.int32).reshape(2, n_chunks_total, CHUNK)

    degp = _make_deg(n_chunks_total)(e2)

    xp = jnp.pad(x, ((0, N_PAD - N_NODES), (0, 0)))
    hp1 = _tc1(degp, xp, W1)

    agg = _make_agg(n_chunks_total)
    s1 = agg(hp1, e2)
    hp2 = _tc2(s1, hp1, degp, b1.reshape(1, HID_DIM), W2)

    s2 = agg(hp2, e2)
    out = _tc3(s2, hp2, degp, b2.reshape(1, HID_DIM), Wf1,
               bf1.reshape(1, HID_DIM), Wf2, bf2.reshape(1, OUT_DIM))
    return out[:N_NODES]


# speedup vs baseline: 1.1097x; 1.0257x over previous
import functools

import jax
import jax.numpy as jnp
from jax import lax
from jax.experimental import pallas as pl
from jax.experimental.pallas import tpu as pltpu
from jax.experimental.pallas import tpu_sc as plsc

N_NODES = 10000
N_PAD = 10240
IN_DIM = 128
HID_DIM = 64
OUT_DIM = 10
NC = 2
NS = 16
NW = NC * NS
CHUNK = 125
NBUF = 5
ROWS_PER_TILE = N_PAD // NS


@functools.lru_cache(maxsize=None)
def _mesh():
    return plsc.VectorSubcoreMesh(core_axis_name="c", subcore_axis_name="s",
                                  num_cores=NC, num_subcores=NS)


NDEG = 4


def _deg_body(e2_hbm, out_hbm, dstv, ones_v, zv, deg_sh, *sems):
    cid = lax.axis_index("c")
    sid = lax.axis_index("s")
    wid = sid * NC + cid
    n_chunks = e2_hbm.shape[1] // NW

    def zstep(i, carry):
        zv[pl.ds(i * 16, 16)] = jnp.zeros((16,), jnp.float32)
        return carry
    lax.fori_loop(0, ROWS_PER_TILE // 16, zstep, 0)
    pltpu.sync_copy(zv, deg_sh.at[pl.ds(sid * ROWS_PER_TILE, ROWS_PER_TILE)])

    def ostep(i, carry):
        ones_v[pl.ds(i * 16, 16)] = jnp.ones((16,), jnp.float32)
        return carry
    lax.fori_loop(0, 8, ostep, 0)

    plsc.subcore_barrier()

    pltpu.sync_copy(e2_hbm.at[1, pl.ds(wid * n_chunks, n_chunks)], dstv)

    def step(j, carry):
        for k in range(NDEG):
            @pl.when(j % NDEG == k)
            def _go():
                @pl.when(j >= NDEG)
                def _wait_prev():
                    pltpu.make_async_copy(
                        ones_v.at[pl.ds(0, CHUNK)],
                        deg_sh.at[dstv.at[j - NDEG]], sems[k]).wait()
                pltpu.async_copy(ones_v.at[pl.ds(0, CHUNK)],
                                 deg_sh.at[dstv.at[j]], sems[k], add=True)
        return carry
    lax.fori_loop(0, n_chunks, step, 0)

    for k in range(NDEG):
        j = n_chunks - NDEG + k
        pltpu.make_async_copy(ones_v.at[pl.ds(0, CHUNK)],
                              deg_sh.at[dstv.at[j]], sems[k % NDEG]).wait()

    plsc.subcore_barrier()
    pltpu.sync_copy(deg_sh.at[pl.ds(sid * ROWS_PER_TILE, ROWS_PER_TILE)],
                    out_hbm.at[cid, pl.ds(sid * ROWS_PER_TILE, ROWS_PER_TILE)])


def _make_deg(n_chunks_total):
    return pl.kernel(
        _deg_body,
        out_type=jax.ShapeDtypeStruct((NC, N_PAD), jnp.float32),
        mesh=_mesh(),
        scratch_types=[
            pltpu.VMEM((n_chunks_total // NW, CHUNK), jnp.int32),
            pltpu.VMEM((128,), jnp.float32),
            pltpu.VMEM((ROWS_PER_TILE,), jnp.float32),
            pltpu.VMEM_SHARED((N_PAD,), jnp.float32),
        ] + [pltpu.SemaphoreType.DMA] * NDEG,
        compiler_params=pltpu.CompilerParams(use_tc_tiling_on_sc=False),
    )


def _agg_body(hp_hbm, e2_hbm, out_hbm, srcv, dstv, *rest):
    rows = list(rest[:NBUF])
    zv = rest[NBUF]
    acc_sh = rest[NBUF + 1]
    sg = list(rest[NBUF + 2:NBUF + 2 + NBUF])
    ss = list(rest[NBUF + 2 + NBUF:])
    cid = lax.axis_index("c")
    sid = lax.axis_index("s")
    wid = sid * NC + cid
    n_chunks = e2_hbm.shape[1] // NW
    n_groups = n_chunks // NBUF

    def zstep(i, carry):
        r = i // 4
        c = i % 4
        zv[r, pl.ds(c * 16, 16)] = jnp.zeros((16,), jnp.float32)
        return carry
    lax.fori_loop(0, 128 * 4, zstep, 0)

    def zcopy(k, carry):
        pltpu.sync_copy(
            zv, acc_sh.at[pl.ds(sid * ROWS_PER_TILE + k * 128, 128)])
        return carry
    lax.fori_loop(0, ROWS_PER_TILE // 128, zcopy, 0)

    plsc.subcore_barrier()

    pltpu.sync_copy(e2_hbm.at[0, pl.ds(wid * n_chunks, n_chunks)], srcv)
    pltpu.sync_copy(e2_hbm.at[1, pl.ds(wid * n_chunks, n_chunks)], dstv)

    for b in range(NBUF):
        pltpu.async_copy(hp_hbm.at[srcv.at[b]], rows[b], sg[b])

    def group(g, carry):
        for b in range(NBUF):
            c = NBUF * g + b
            pltpu.make_async_copy(hp_hbm.at[srcv.at[c]], rows[b],
                                  sg[b]).wait()
            pltpu.async_copy(rows[b], acc_sh.at[dstv.at[c]], ss[b], add=True)
        for b in range(NBUF):
            c = NBUF * g + b

            @pl.when(g < n_groups - 1)
            def _next():
                pltpu.make_async_copy(rows[b], acc_sh.at[dstv.at[c]],
                                      ss[b]).wait()
                pltpu.async_copy(hp_hbm.at[srcv.at[c + NBUF]], rows[b], sg[b])
        return carry
    lax.fori_loop(0, n_groups, group, 0)

    for b in range(NBUF):
        pltpu.make_async_copy(rows[b],
                              acc_sh.at[dstv.at[n_chunks - NBUF + b]],
                              ss[b]).wait()

    plsc.subcore_barrier()
    pltpu.sync_copy(
        acc_sh.at[pl.ds(sid * ROWS_PER_TILE, ROWS_PER_TILE)],
        out_hbm.at[cid, pl.ds(sid * ROWS_PER_TILE, ROWS_PER_TILE)])


def _make_agg(n_chunks_total):
    return pl.kernel(
        _agg_body,
        out_type=jax.ShapeDtypeStruct((NC, N_PAD, HID_DIM), jnp.float32),
        mesh=_mesh(),
        scratch_types=(
            [pltpu.VMEM((n_chunks_total // NW, CHUNK), jnp.int32)] * 2
            + [pltpu.VMEM((CHUNK, HID_DIM), jnp.float32)] * NBUF
            + [pltpu.VMEM((128, HID_DIM), jnp.float32),
               pltpu.VMEM_SHARED((N_PAD, HID_DIM), jnp.float32)]
            + [pltpu.SemaphoreType.DMA] * (2 * NBUF)
        ),
        compiler_params=pltpu.CompilerParams(use_tc_tiling_on_sc=False),
    )


_BR = 2048


def _dinv(deg_ref):
    deg = deg_ref[0] + deg_ref[1] + 1.0
    return lax.rsqrt(deg)[:, None]


def _tc1_body(deg_ref, x_ref, w1_ref, hp1_ref):
    h = jnp.dot(x_ref[...], w1_ref[...], preferred_element_type=jnp.float32)
    hp1_ref[...] = h * _dinv(deg_ref)


def _tc2_body(s_ref, hp1_ref, deg_ref, b1_ref, w2_ref, hp2_ref):
    dinv = _dinv(deg_ref)
    s = s_ref[0] + s_ref[1] + hp1_ref[...]
    out1 = jnp.maximum(dinv * s + b1_ref[...], 0.0)
    h2 = jnp.dot(out1, w2_ref[...], preferred_element_type=jnp.float32)
    hp2_ref[...] = h2 * dinv


def _tc3_body(s_ref, hp2_ref, deg_ref, b2_ref, wf1_ref, bf1_ref,
              wf2_ref, bf2_ref, out_ref):
    dinv = _dinv(deg_ref)
    s = s_ref[0] + s_ref[1] + hp2_ref[...]
    out2 = jnp.maximum(dinv * s + b2_ref[...], 0.0)
    h3 = jnp.maximum(
        jnp.dot(out2, wf1_ref[...], preferred_element_type=jnp.float32)
        + bf1_ref[...], 0.0)
    logits = (jnp.dot(h3, wf2_ref[...], preferred_element_type=jnp.float32)
              + bf2_ref[...])
    m = jnp.max(logits, axis=1, keepdims=True)
    e = jnp.exp(logits - m)
    out_ref[...] = e / jnp.sum(e, axis=1, keepdims=True)


def _row_spec(cols):
    return pl.BlockSpec((_BR, cols), lambda i: (i, 0))


def _pad_spec(cols):
    return pl.BlockSpec((NC, _BR, cols), lambda i: (0, i, 0))


_deg_spec = pl.BlockSpec((NC, _BR), lambda i: (0, i))


def _full_spec(shape):
    nd = len(shape)
    return pl.BlockSpec(shape, lambda i: (0,) * nd)


_GRID = (N_PAD // _BR,)

_tc1 = pl.pallas_call(
    _tc1_body,
    grid=_GRID,
    in_specs=[
        _deg_spec,
        _row_spec(IN_DIM),
        _full_spec((IN_DIM, HID_DIM)),
    ],
    out_specs=_row_spec(HID_DIM),
    out_shape=jax.ShapeDtypeStruct((N_PAD, HID_DIM), jnp.float32),
)

_tc2 = pl.pallas_call(
    _tc2_body,
    grid=_GRID,
    in_specs=[
        _pad_spec(HID_DIM),
        _row_spec(HID_DIM),
        _deg_spec,
        _full_spec((1, HID_DIM)),
        _full_spec((HID_DIM, HID_DIM)),
    ],
    out_specs=_row_spec(HID_DIM),
    out_shape=jax.ShapeDtypeStruct((N_PAD, HID_DIM), jnp.float32),
)

_tc3 = pl.pallas_call(
    _tc3_body,
    grid=_GRID,
    in_specs=[
        _pad_spec(HID_DIM),
        _row_spec(HID_DIM),
        _deg_spec,
        _full_spec((1, HID_DIM)),
        _full_spec((HID_DIM, HID_DIM)),
        _full_spec((1, HID_DIM)),
        _full_spec((HID_DIM, OUT_DIM)),
        _full_spec((1, OUT_DIM)),
    ],
    out_specs=_row_spec(OUT_DIM),
    out_shape=jax.ShapeDtypeStruct((N_PAD, OUT_DIM), jnp.float32),
)


@jax.jit
def kernel(x, edge_index, W1, b1, W2, b2, Wf1, bf1, Wf2, bf2):
    n_edges = edge_index.shape[1]
    n_chunks_total = n_edges // CHUNK
    e2 = edge_index.astype(jnp.int32).reshape(2, n_chunks_total, CHUNK)

    degp = _make_deg(n_chunks_total)(e2)

    xp = jnp.pad(x, ((0, N_PAD - N_NODES), (0, 0)))
    hp1 = _tc1(degp, xp, W1)

    agg = _make_agg(n_chunks_total)
    s1 = agg(hp1, e2)
    hp2 = _tc2(s1, hp1, degp, b1.reshape(1, HID_DIM), W2)

    s2 = agg(hp2, e2)
    out = _tc3(s2, hp2, degp, b2.reshape(1, HID_DIM), Wf1,
               bf1.reshape(1, HID_DIM), Wf2, bf2.reshape(1, OUT_DIM))
    return out[:N_NODES]
